# Initial kernel scaffold; baseline (speedup 1.0000x reference)
import jax
import jax.numpy as jnp
from jax.experimental import pallas as pl


def kernel(f_a, position_embedding, iou, WG_w, WG_b, WK_w, WK_b, WQ_w, WQ_b, conv_w, conv_b):
    raise NotImplementedError("write your pallas kernel here")


# speedup vs baseline: 11.9768x; 11.9768x over previous
import functools

import jax
import jax.numpy as jnp
import numpy as np
from jax.experimental import pallas as pl

_G = 16
_N = 1000
_NP = 1024
_F = 1024
_M = 8


def _proj_kernel(fa_ref, w_ref, b_ref, o_ref):
    o_ref[0] = jax.lax.dot_general(
        fa_ref[0], w_ref[...], (((1,), (1,)), ((), ())),
        preferred_element_type=jnp.float32) + b_ref[...]


def _main_kernel(pe_ref, iou_ref, q_ref, k_ref, p_ref, wgw_ref, wgb_ref,
                 cb_ref, y_ref):
    M = pe_ref.shape[1]
    pe = pe_ref[0]
    pe = jnp.concatenate(
        [pe, jnp.zeros((M, _NP - _N, 64), jnp.float32)], axis=1)
    pe2 = pe.reshape(M * _NP, 64)
    wgt = jax.lax.dot_general(
        wgw_ref[...], pe2, (((1,), (1,)), ((), ())),
        preferred_element_type=jnp.float32) + wgb_ref[...]
    lg = jnp.log(jnp.maximum(wgt, 1e-6)).reshape(_G * M, _NP)

    iou = iou_ref[0]
    logc = jnp.log(jnp.asarray(1e-6, jnp.float32))
    liou = jnp.where(iou >= 1e-6, jnp.asarray(0.0, jnp.float32), logc)

    qblk = q_ref[0]
    k2 = k_ref[0]
    affs = []
    for j in range(_G):
        qj = qblk[:, j * 64:(j + 1) * 64]
        kj = k2[j * 64:(j + 1) * 64, :]
        affs.append(jax.lax.dot_general(
            qj, kj, (((1,), (0,)), ((), ())),
            preferred_element_type=jnp.float32) * 0.125 + liou)
    w = lg + jnp.concatenate(affs, axis=0)

    iota = jax.lax.broadcasted_iota(jnp.int32, (1, _NP), 1)
    w = jnp.where(iota < _N, w, -jnp.inf)

    acc = jnp.zeros((_G * M, _NP), jnp.float32)
    m0 = None
    z = None
    for t in range(10):
        mx = jnp.max(w, axis=1, keepdims=True)
        if t == 0:
            m0 = mx
            e = jnp.ones_like(mx)
            z = e
        else:
            e = jnp.exp(mx - m0)
            z = z + e
        sel = jnp.where(w == mx, iota, _NP + 1)
        first = jnp.min(sel, axis=1, keepdims=True)
        oh = iota == first
        acc = acc + jnp.where(oh, e, jnp.asarray(0.0, jnp.float32))
        w = jnp.where(oh, -jnp.inf, w)
    wsp = acc / z

    pp = p_ref[0]
    outs = []
    for j in range(_G):
        wj = wsp[j * M:(j + 1) * M, :]
        pj = pp[:, j * 64:(j + 1) * 64]
        outs.append(jax.lax.dot_general(
            wj, pj, (((1,), (0,)), ((), ())),
            preferred_element_type=jnp.float32))
    y_ref[0] = jnp.concatenate(outs, axis=1) + cb_ref[...]


@jax.jit
def kernel(f_a, position_embedding, iou, WG_w, WG_b, WK_w, WK_b, WQ_w, WQ_b,
           conv_w, conv_b):
    N, C, F = f_a.shape
    fa = jnp.transpose(f_a, (1, 0, 2))
    conv2d = conv_w[:, :, 0, 0]
    wcat = jnp.concatenate([WQ_w, WK_w, conv2d], axis=0)
    bcat = jnp.concatenate(
        [WQ_b, WK_b, jnp.zeros_like(conv_b)])[None, :]

    qkp = pl.pallas_call(
        _proj_kernel,
        grid=(C, 3),
        in_specs=[
            pl.BlockSpec((1, N, F), lambda c, t: (c, 0, 0)),
            pl.BlockSpec((F, F), lambda c, t: (t, 0)),
            pl.BlockSpec((1, F), lambda c, t: (0, t)),
        ],
        out_specs=pl.BlockSpec((1, N, F), lambda c, t: (c, 0, t)),
        out_shape=jax.ShapeDtypeStruct((C, N, 3 * F), jnp.float32),
    )(fa, wcat, bcat)

    q = qkp[:, :, :F]
    kt = jnp.pad(jnp.transpose(qkp[:, :, F:2 * F], (0, 2, 1)),
                 ((0, 0), (0, 0), (0, _NP - N)))
    ppad = jnp.pad(qkp[:, :, 2 * F:],
                   ((0, 0), (0, _NP - N), (0, 0)))
    ioup = jnp.pad(iou, ((0, 0), (0, 0), (0, _NP - N)))

    y = pl.pallas_call(
        _main_kernel,
        grid=(C, N // _M),
        in_specs=[
            pl.BlockSpec((1, _M, N, 64), lambda c, i: (c, i, 0, 0)),
            pl.BlockSpec((1, _M, _NP), lambda c, i: (c, i, 0)),
            pl.BlockSpec((1, _M, F), lambda c, i: (c, i, 0)),
            pl.BlockSpec((1, F, _NP), lambda c, i: (c, 0, 0)),
            pl.BlockSpec((1, _NP, F), lambda c, i: (c, 0, 0)),
            pl.BlockSpec((_G, 64), lambda c, i: (0, 0)),
            pl.BlockSpec((_G, 1), lambda c, i: (0, 0)),
            pl.BlockSpec((1, F), lambda c, i: (0, 0)),
        ],
        out_specs=pl.BlockSpec((1, _M, F), lambda c, i: (c, i, 0)),
        out_shape=jax.ShapeDtypeStruct((C, N, F), jnp.float32),
    )(position_embedding, ioup, q, kt, ppad, WG_w, WG_b[:, None],
      conv_b[None, :])

    return jnp.transpose(y, (1, 0, 2))
